# Initial kernel scaffold; baseline (speedup 1.0000x reference)
import jax
import jax.numpy as jnp
from jax.experimental import pallas as pl


def kernel(x, edge_index, lin_w, film_w, film_b, lin_skip_w, film_skip_w):
    raise NotImplementedError("write your pallas kernel here")


# speedup vs baseline: 3.0226x; 3.0226x over previous
import functools

import jax
import jax.numpy as jnp
from jax import lax
from jax.experimental import pallas as pl
from jax.experimental.pallas import tpu as pltpu
from jax.experimental.pallas import tpu_sc as plsc

N_NODES = 10000
CIN = 128
COUT = 128
N_EDGES = 320000

NC, NS = 2, 16
NW = NC * NS
EPW = N_EDGES // NW
B = 80
NB = EPW // B
ZCH = 80
NCHUNK = N_NODES // ZCH
CPT = (NCHUNK + NS - 1) // NS

ROWS_TC = 1000
GRID_TC = N_NODES // ROWS_TC


def _leaky(t):
    return jnp.maximum(t, 0.01 * t)


def _dense_body(x_ref, lw_ref, fw_ref, fb_ref, lsw_ref, fsw_ref,
                xl_ref, fg_ref, outk_ref):
    x = x_ref[...]
    xl_ref[...] = jnp.dot(x, lw_ref[...], preferred_element_type=jnp.float32)
    fg_ref[...] = (jnp.dot(x, fw_ref[...], preferred_element_type=jnp.float32)
                   + fb_ref[...])
    fgs = jnp.dot(x, fsw_ref[...], preferred_element_type=jnp.float32)
    ls = jnp.dot(x, lsw_ref[...], preferred_element_type=jnp.float32)
    outk_ref[...] = _leaky(fgs[:, COUT:] * ls + fgs[:, :COUT])


def _dense(x, lin_w, film_w, film_b, lin_skip_w, film_skip_w):
    return pl.pallas_call(
        _dense_body,
        grid=(GRID_TC,),
        in_specs=[
            pl.BlockSpec((ROWS_TC, CIN), lambda i: (i, 0)),
            pl.BlockSpec((CIN, COUT), lambda i: (0, 0)),
            pl.BlockSpec((CIN, 2 * COUT), lambda i: (0, 0)),
            pl.BlockSpec((1, 2 * COUT), lambda i: (0, 0)),
            pl.BlockSpec((CIN, COUT), lambda i: (0, 0)),
            pl.BlockSpec((CIN, 2 * COUT), lambda i: (0, 0)),
        ],
        out_specs=[
            pl.BlockSpec((ROWS_TC, COUT), lambda i: (i, 0)),
            pl.BlockSpec((ROWS_TC, 2 * COUT), lambda i: (i, 0)),
            pl.BlockSpec((ROWS_TC, COUT), lambda i: (i, 0)),
        ],
        out_shape=[
            jax.ShapeDtypeStruct((N_NODES, COUT), jnp.float32),
            jax.ShapeDtypeStruct((N_NODES, 2 * COUT), jnp.float32),
            jax.ShapeDtypeStruct((N_NODES, COUT), jnp.float32),
        ],
    )(x, lin_w, film_w, film_b.reshape(1, 2 * COUT), lin_skip_w, film_skip_w)


def _edge_body(src_hbm, dst_hbm, xl_hbm, fg_hbm, out_hbm, cnt_hbm,
               srcb, dstb, xlb, fgb, msgb, acc, sem0, sem1):
    cid = lax.axis_index("c")
    sid = lax.axis_index("s")
    wid = cid * NS + sid
    ebase = wid * EPW

    def _fill(ref, val):
        def _row(r, _):
            for c in range(COUT // 16):
                ref[r, pl.ds(c * 16, 16)] = jnp.full((16,), val, jnp.float32)
            return 0
        lax.fori_loop(0, B, _row, 0)

    def _zero_acc():
        for k in range(CPT):
            ch = sid + NS * k
            @pl.when(ch < NCHUNK)
            def _():
                pltpu.sync_copy(msgb, acc.at[pl.ds(ch * ZCH, ZCH)])

    def _publish(dst_ref):
        for k in range(CPT):
            ch = sid + NS * k
            @pl.when(ch < NCHUNK)
            def _():
                pltpu.sync_copy(acc.at[pl.ds(ch * ZCH, ZCH)],
                                dst_ref.at[cid, pl.ds(ch * ZCH, ZCH)])

    _fill(msgb, 0.0)
    _zero_acc()
    plsc.subcore_barrier()

    def _batch(k, _):
        base = ebase + k * B
        pltpu.sync_copy(src_hbm.at[pl.ds(base, B)], srcb)
        pltpu.sync_copy(dst_hbm.at[pl.ds(base, B)], dstb)
        h0 = pltpu.async_copy(xl_hbm.at[srcb], xlb, sem0)
        h1 = pltpu.async_copy(fg_hbm.at[dstb], fgb, sem1)
        h0.wait()
        h1.wait()

        def _edge(e, _):
            for c in range(COUT // 16):
                xv = xlb[e, pl.ds(c * 16, 16)]
                bv = fgb[e, pl.ds(c * 16, 16)]
                gv = fgb[e, pl.ds(COUT + c * 16, 16)]
                msgb[e, pl.ds(c * 16, 16)] = _leaky(gv * xv + bv)
            return 0
        lax.fori_loop(0, B, _edge, 0)

        pltpu.sync_copy(msgb, acc.at[dstb], add=True)
        return 0
    lax.fori_loop(0, NB, _batch, 0)

    plsc.subcore_barrier()
    _publish(out_hbm)
    plsc.subcore_barrier()

    _fill(msgb, 0.0)
    _zero_acc()
    plsc.subcore_barrier()
    _fill(msgb, 1.0)

    def _cbatch(k, _):
        base = ebase + k * B
        pltpu.sync_copy(dst_hbm.at[pl.ds(base, B)], dstb)
        pltpu.sync_copy(msgb, acc.at[dstb], add=True)
        return 0
    lax.fori_loop(0, NB, _cbatch, 0)

    plsc.subcore_barrier()
    _publish(cnt_hbm)


def _edge_pass(src, dst, xl, fg):
    mesh = plsc.VectorSubcoreMesh(core_axis_name="c", subcore_axis_name="s",
                                  num_cores=NC, num_subcores=NS)
    f = functools.partial(
        pl.kernel,
        out_type=(
            jax.ShapeDtypeStruct((NC, N_NODES, COUT), jnp.float32),
            jax.ShapeDtypeStruct((NC, N_NODES, COUT), jnp.float32),
        ),
        mesh=mesh,
        scratch_types=[
            pltpu.VMEM((B,), jnp.int32),
            pltpu.VMEM((B,), jnp.int32),
            pltpu.VMEM((B, CIN), jnp.float32),
            pltpu.VMEM((B, 2 * COUT), jnp.float32),
            pltpu.VMEM((B, COUT), jnp.float32),
            pltpu.VMEM_SHARED((N_NODES, COUT), jnp.float32),
            pltpu.SemaphoreType.DMA,
            pltpu.SemaphoreType.DMA,
        ],
    )(_edge_body)
    return f(src, dst, xl, fg)


def _combine_body(acc_ref, cnt_ref, outk_ref, o_ref):
    agg = acc_ref[0] + acc_ref[1]
    cnt = cnt_ref[0, :, :1] + cnt_ref[1, :, :1]
    o_ref[...] = outk_ref[...] + agg / jnp.maximum(cnt, 1.0)


def _combine(acc, cnt, outk):
    return pl.pallas_call(
        _combine_body,
        out_shape=jax.ShapeDtypeStruct((N_NODES, COUT), jnp.float32),
    )(acc, cnt, outk)


def kernel(x, edge_index, lin_w, film_w, film_b, lin_skip_w, film_skip_w):
    src = edge_index[0].astype(jnp.int32)
    dst = edge_index[1].astype(jnp.int32)
    xl, fg, outk = _dense(x, lin_w, film_w, film_b, lin_skip_w, film_skip_w)
    acc, cnt = _edge_pass(src, dst, xl, fg)
    return _combine(acc, cnt, outk)


# speedup vs baseline: 4.5768x; 1.5142x over previous
import functools

import jax
import jax.numpy as jnp
from jax import lax
from jax.experimental import pallas as pl
from jax.experimental.pallas import tpu as pltpu
from jax.experimental.pallas import tpu_sc as plsc

N_NODES = 10000
CIN = 128
COUT = 128
N_EDGES = 320000

NC, NS = 2, 16
NW = NC * NS
EPW = N_EDGES // NW
B = 40
NB = EPW // B
BPB = 10
NBLK = NB // BPB
ZCH = B
NCHUNK = N_NODES // ZCH
CPT = (NCHUNK + NS - 1) // NS

ROWS_TC = 1000
GRID_TC = N_NODES // ROWS_TC

EB = 20000
GRID_H = N_EDGES // EB


def _leaky(t):
    return jnp.maximum(t, 0.01 * t)


def _dense_body(x_ref, lw_ref, fw_ref, fb_ref, lsw_ref, fsw_ref,
                xl_ref, fg_ref, outk_ref):
    x = x_ref[...]
    xl_ref[...] = jnp.dot(x, lw_ref[...], preferred_element_type=jnp.float32)
    fg_ref[...] = (jnp.dot(x, fw_ref[...], preferred_element_type=jnp.float32)
                   + fb_ref[...])
    fgs = jnp.dot(x, fsw_ref[...], preferred_element_type=jnp.float32)
    ls = jnp.dot(x, lsw_ref[...], preferred_element_type=jnp.float32)
    outk_ref[...] = _leaky(fgs[:, COUT:] * ls + fgs[:, :COUT])


def _dense(x, lin_w, film_w, film_b, lin_skip_w, film_skip_w):
    return pl.pallas_call(
        _dense_body,
        grid=(GRID_TC,),
        in_specs=[
            pl.BlockSpec((ROWS_TC, CIN), lambda i: (i, 0)),
            pl.BlockSpec((CIN, COUT), lambda i: (0, 0)),
            pl.BlockSpec((CIN, 2 * COUT), lambda i: (0, 0)),
            pl.BlockSpec((1, 2 * COUT), lambda i: (0, 0)),
            pl.BlockSpec((CIN, COUT), lambda i: (0, 0)),
            pl.BlockSpec((CIN, 2 * COUT), lambda i: (0, 0)),
        ],
        out_specs=[
            pl.BlockSpec((ROWS_TC, COUT), lambda i: (i, 0)),
            pl.BlockSpec((ROWS_TC, 2 * COUT), lambda i: (i, 0)),
            pl.BlockSpec((ROWS_TC, COUT), lambda i: (i, 0)),
        ],
        out_shape=[
            jax.ShapeDtypeStruct((N_NODES, COUT), jnp.float32),
            jax.ShapeDtypeStruct((N_NODES, 2 * COUT), jnp.float32),
            jax.ShapeDtypeStruct((N_NODES, COUT), jnp.float32),
        ],
    )(x, lin_w, film_w, film_b.reshape(1, 2 * COUT), lin_skip_w, film_skip_w)


def _hist_body(dst_ref, h_ref):
    i = pl.program_id(0)
    d = dst_ref[...]
    hi = d >> 7
    lo = d & 127
    cols = lax.broadcasted_iota(jnp.int32, (1, 128), 1)
    oh_hi = (hi == cols).astype(jnp.bfloat16)
    oh_lo = (lo == cols).astype(jnp.bfloat16)
    prod = lax.dot_general(oh_hi, oh_lo, (((0,), (0,)), ((), ())),
                           preferred_element_type=jnp.float32)

    @pl.when(i == 0)
    def _():
        h_ref[...] = jnp.zeros_like(h_ref)
    h_ref[...] += prod


def _degree(dst2d):
    return pl.pallas_call(
        _hist_body,
        grid=(GRID_H,),
        in_specs=[pl.BlockSpec((EB, 1), lambda i: (i, 0))],
        out_specs=pl.BlockSpec((128, 128), lambda i: (0, 0)),
        out_shape=jax.ShapeDtypeStruct((128, 128), jnp.float32),
    )(dst2d)


def _edge_body(src_hbm, dst_hbm, xl_hbm, fg_hbm, out_hbm,
               srcblk, dstblk, xlb, fgb, msgb, acc,
               semg0, semg1, sems0, sems1):
    cid = lax.axis_index("c")
    sid = lax.axis_index("s")
    wid = cid * NS + sid

    def _zero_acc():
        def _row(r, _):
            for c in range(COUT // 16):
                msgb[0, r, pl.ds(c * 16, 16)] = jnp.zeros((16,), jnp.float32)
            return 0
        lax.fori_loop(0, B, _row, 0)
        for k in range(CPT):
            ch = sid + NS * k
            @pl.when(ch < NCHUNK)
            def _():
                pltpu.sync_copy(msgb.at[0], acc.at[pl.ds(ch * ZCH, ZCH)])

    _zero_acc()
    plsc.subcore_barrier()

    def _block(blk, _):
        @pl.when(blk > 0)
        def _():
            pltpu.make_async_copy(msgb.at[0], acc.at[dstblk.at[0]],
                                  sems0).wait()
            pltpu.make_async_copy(msgb.at[1], acc.at[dstblk.at[0]],
                                  sems1).wait()
        pltpu.sync_copy(src_hbm.at[wid, blk], srcblk)
        pltpu.sync_copy(dst_hbm.at[wid, blk], dstblk)
        pltpu.async_copy(xl_hbm.at[srcblk.at[0]], xlb.at[0], semg0)
        pltpu.async_copy(fg_hbm.at[dstblk.at[0]], fgb.at[0], semg0)

        def _pair(p, _):
            for s in (0, 1):
                j = 2 * p + s
                mine = semg0 if s == 0 else semg1
                other = semg1 if s == 0 else semg0
                scs = sems0 if s == 0 else sems1
                pltpu.make_async_copy(xl_hbm.at[srcblk.at[0]],
                                      xlb.at[s], mine).wait()
                pltpu.make_async_copy(fg_hbm.at[dstblk.at[0]],
                                      fgb.at[s], mine).wait()

                @pl.when(j + 1 < BPB)
                def _():
                    pltpu.async_copy(xl_hbm.at[srcblk.at[j + 1]],
                                     xlb.at[1 - s], other)
                    pltpu.async_copy(fg_hbm.at[dstblk.at[j + 1]],
                                     fgb.at[1 - s], other)

                @pl.when(p >= 1)
                def _():
                    pltpu.make_async_copy(msgb.at[s], acc.at[dstblk.at[0]],
                                          scs).wait()

                def _edge(e, _):
                    for c in range(COUT // 16):
                        xv = xlb[s, e, pl.ds(c * 16, 16)]
                        bv = fgb[s, e, pl.ds(c * 16, 16)]
                        gv = fgb[s, e, pl.ds(COUT + c * 16, 16)]
                        msgb[s, e, pl.ds(c * 16, 16)] = _leaky(gv * xv + bv)
                    return 0
                lax.fori_loop(0, B, _edge, 0)

                pltpu.async_copy(msgb.at[s], acc.at[dstblk.at[j]], scs,
                                 add=True)
            return 0
        lax.fori_loop(0, BPB // 2, _pair, 0)
        return 0
    lax.fori_loop(0, NBLK, _block, 0)

    pltpu.make_async_copy(msgb.at[0], acc.at[dstblk.at[0]], sems0).wait()
    pltpu.make_async_copy(msgb.at[1], acc.at[dstblk.at[0]], sems1).wait()
    plsc.subcore_barrier()

    for k in range(CPT):
        ch = sid + NS * k
        @pl.when(ch < NCHUNK)
        def _():
            pltpu.sync_copy(acc.at[pl.ds(ch * ZCH, ZCH)],
                            out_hbm.at[cid, pl.ds(ch * ZCH, ZCH)])


def _edge_pass(src, dst, xl, fg):
    mesh = plsc.VectorSubcoreMesh(core_axis_name="c", subcore_axis_name="s",
                                  num_cores=NC, num_subcores=NS)
    f = functools.partial(
        pl.kernel,
        out_type=jax.ShapeDtypeStruct((NC, N_NODES, COUT), jnp.float32),
        mesh=mesh,
        scratch_types=[
            pltpu.VMEM((BPB, B), jnp.int32),
            pltpu.VMEM((BPB, B), jnp.int32),
            pltpu.VMEM((2, B, CIN), jnp.float32),
            pltpu.VMEM((2, B, 2 * COUT), jnp.float32),
            pltpu.VMEM((2, B, COUT), jnp.float32),
            pltpu.VMEM_SHARED((N_NODES, COUT), jnp.float32),
            pltpu.SemaphoreType.DMA,
            pltpu.SemaphoreType.DMA,
            pltpu.SemaphoreType.DMA,
            pltpu.SemaphoreType.DMA,
        ],
    )(_edge_body)
    return f(src.reshape(NW, NBLK, BPB, B), dst.reshape(NW, NBLK, BPB, B),
             xl, fg)


def _combine_body(acc_ref, cnt_ref, outk_ref, o_ref):
    agg = acc_ref[0] + acc_ref[1]
    o_ref[...] = outk_ref[...] + agg / jnp.maximum(cnt_ref[...], 1.0)


def _combine(acc, cnt, outk):
    return pl.pallas_call(
        _combine_body,
        out_shape=jax.ShapeDtypeStruct((N_NODES, COUT), jnp.float32),
    )(acc, cnt, outk)


def kernel(x, edge_index, lin_w, film_w, film_b, lin_skip_w, film_skip_w):
    src = edge_index[0].astype(jnp.int32)
    dst = edge_index[1].astype(jnp.int32)
    xl, fg, outk = _dense(x, lin_w, film_w, film_b, lin_skip_w, film_skip_w)
    acc = _edge_pass(src, dst, xl, fg)
    hist = _degree(dst.reshape(N_EDGES, 1))
    cnt = hist.reshape(-1)[:N_NODES].reshape(N_NODES, 1)
    return _combine(acc, cnt, outk)


# speedup vs baseline: 4.8464x; 1.0589x over previous
import functools

import jax
import jax.numpy as jnp
from jax import lax
from jax.experimental import pallas as pl
from jax.experimental.pallas import tpu as pltpu
from jax.experimental.pallas import tpu_sc as plsc

N_NODES = 10000
CIN = 128
COUT = 128
N_EDGES = 320000

NC, NS = 2, 16
NW = NC * NS
EPW = N_EDGES // NW
B = 40
NB = EPW // B
BPB = 25
NBLK = NB // BPB
ZCH = B
NCHUNK = N_NODES // ZCH
CPT = (NCHUNK + NS - 1) // NS

ROWS_TC = 1000
GRID_TC = N_NODES // ROWS_TC

EB = 20000
GRID_H = N_EDGES // EB


def _leaky(t):
    return jnp.maximum(t, 0.01 * t)


def _dense_body(x_ref, lw_ref, fw_ref, fb_ref, lsw_ref, fsw_ref,
                xl_ref, fg_ref, outk_ref):
    x = x_ref[...]
    xl_ref[...] = jnp.dot(x, lw_ref[...], preferred_element_type=jnp.float32)
    fg_ref[...] = (jnp.dot(x, fw_ref[...], preferred_element_type=jnp.float32)
                   + fb_ref[...])
    fgs = jnp.dot(x, fsw_ref[...], preferred_element_type=jnp.float32)
    ls = jnp.dot(x, lsw_ref[...], preferred_element_type=jnp.float32)
    outk_ref[...] = _leaky(fgs[:, COUT:] * ls + fgs[:, :COUT])


def _dense(x, lin_w, film_w, film_b, lin_skip_w, film_skip_w):
    return pl.pallas_call(
        _dense_body,
        grid=(GRID_TC,),
        in_specs=[
            pl.BlockSpec((ROWS_TC, CIN), lambda i: (i, 0)),
            pl.BlockSpec((CIN, COUT), lambda i: (0, 0)),
            pl.BlockSpec((CIN, 2 * COUT), lambda i: (0, 0)),
            pl.BlockSpec((1, 2 * COUT), lambda i: (0, 0)),
            pl.BlockSpec((CIN, COUT), lambda i: (0, 0)),
            pl.BlockSpec((CIN, 2 * COUT), lambda i: (0, 0)),
        ],
        out_specs=[
            pl.BlockSpec((ROWS_TC, COUT), lambda i: (i, 0)),
            pl.BlockSpec((ROWS_TC, 2 * COUT), lambda i: (i, 0)),
            pl.BlockSpec((ROWS_TC, COUT), lambda i: (i, 0)),
        ],
        out_shape=[
            jax.ShapeDtypeStruct((N_NODES, COUT), jnp.float32),
            jax.ShapeDtypeStruct((N_NODES, 2 * COUT), jnp.float32),
            jax.ShapeDtypeStruct((N_NODES, COUT), jnp.float32),
        ],
    )(x, lin_w, film_w, film_b.reshape(1, 2 * COUT), lin_skip_w, film_skip_w)


def _hist_body(dst_ref, h_ref):
    i = pl.program_id(0)
    d = dst_ref[...]
    hi = d >> 7
    lo = d & 127
    cols = lax.broadcasted_iota(jnp.int32, (1, 128), 1)
    oh_hi = (hi == cols).astype(jnp.bfloat16)
    oh_lo = (lo == cols).astype(jnp.bfloat16)
    prod = lax.dot_general(oh_hi, oh_lo, (((0,), (0,)), ((), ())),
                           preferred_element_type=jnp.float32)

    @pl.when(i == 0)
    def _():
        h_ref[...] = jnp.zeros_like(h_ref)
    h_ref[...] += prod


def _degree(dst2d):
    return pl.pallas_call(
        _hist_body,
        grid=(GRID_H,),
        in_specs=[pl.BlockSpec((EB, 1), lambda i: (i, 0))],
        out_specs=pl.BlockSpec((128, 128), lambda i: (0, 0)),
        out_shape=jax.ShapeDtypeStruct((128, 128), jnp.float32),
    )(dst2d)


def _edge_body(src_hbm, dst_hbm, xl_hbm, fg_hbm, out_hbm,
               srcblk, dstblk, xlb, fgb, msgb, acc,
               semg0, semg1, sems0, sems1):
    cid = lax.axis_index("c")
    sid = lax.axis_index("s")
    wid = cid * NS + sid

    def _zero_acc():
        def _row(r, _):
            for c in range(COUT // 16):
                msgb[0, r, pl.ds(c * 16, 16)] = jnp.zeros((16,), jnp.float32)
            return 0
        lax.fori_loop(0, B, _row, 0)
        for k in range(CPT):
            ch = sid + NS * k
            @pl.when(ch < NCHUNK)
            def _():
                pltpu.async_copy(msgb.at[0], acc.at[pl.ds(ch * ZCH, ZCH)],
                                 semg0)
        for k in range(CPT):
            ch = sid + NS * k
            @pl.when(ch < NCHUNK)
            def _():
                pltpu.make_async_copy(msgb.at[0],
                                      acc.at[pl.ds(ch * ZCH, ZCH)],
                                      semg0).wait()

    _zero_acc()
    plsc.subcore_barrier()

    def _block(blk, _):
        @pl.when(blk > 0)
        def _():
            pltpu.make_async_copy(msgb.at[0], acc.at[dstblk.at[0]],
                                  sems0).wait()
            pltpu.make_async_copy(msgb.at[1], acc.at[dstblk.at[0]],
                                  sems1).wait()
        pltpu.sync_copy(src_hbm.at[wid, blk], srcblk)
        pltpu.sync_copy(dst_hbm.at[wid, blk], dstblk)
        pltpu.async_copy(xl_hbm.at[srcblk.at[0]], xlb.at[0], semg0)
        pltpu.async_copy(fg_hbm.at[dstblk.at[0]], fgb.at[0], semg0)

        def _pair(p, _):
            for s in (0, 1):
                j = 2 * p + s
                mine = semg0 if s == 0 else semg1
                other = semg1 if s == 0 else semg0
                scs = sems0 if s == 0 else sems1
                pltpu.make_async_copy(xl_hbm.at[srcblk.at[0]],
                                      xlb.at[s], mine).wait()
                pltpu.make_async_copy(fg_hbm.at[dstblk.at[0]],
                                      fgb.at[s], mine).wait()

                @pl.when(j + 1 < BPB)
                def _():
                    pltpu.async_copy(xl_hbm.at[srcblk.at[j + 1]],
                                     xlb.at[1 - s], other)
                    pltpu.async_copy(fg_hbm.at[dstblk.at[j + 1]],
                                     fgb.at[1 - s], other)

                @pl.when(p >= 1)
                def _():
                    pltpu.make_async_copy(msgb.at[s], acc.at[dstblk.at[0]],
                                          scs).wait()

                def _edge(e2, _):
                    for u in (0, 1):
                        e = 2 * e2 + u
                        for c in range(COUT // 16):
                            xv = xlb[s, e, pl.ds(c * 16, 16)]
                            bv = fgb[s, e, pl.ds(c * 16, 16)]
                            gv = fgb[s, e, pl.ds(COUT + c * 16, 16)]
                            msgb[s, e, pl.ds(c * 16, 16)] = _leaky(gv * xv + bv)
                    return 0
                lax.fori_loop(0, B // 2, _edge, 0)

                pltpu.async_copy(msgb.at[s], acc.at[dstblk.at[j]], scs,
                                 add=True)
            return 0
        lax.fori_loop(0, BPB // 2, _pair, 0)

        jt = BPB - 1
        pltpu.make_async_copy(xl_hbm.at[srcblk.at[0]],
                              xlb.at[0], semg0).wait()
        pltpu.make_async_copy(fg_hbm.at[dstblk.at[0]],
                              fgb.at[0], semg0).wait()
        pltpu.make_async_copy(msgb.at[0], acc.at[dstblk.at[0]],
                              sems0).wait()

        def _edge_t(e2, _):
            for u in (0, 1):
                e = 2 * e2 + u
                for c in range(COUT // 16):
                    xv = xlb[0, e, pl.ds(c * 16, 16)]
                    bv = fgb[0, e, pl.ds(c * 16, 16)]
                    gv = fgb[0, e, pl.ds(COUT + c * 16, 16)]
                    msgb[0, e, pl.ds(c * 16, 16)] = _leaky(gv * xv + bv)
            return 0
        lax.fori_loop(0, B // 2, _edge_t, 0)

        pltpu.async_copy(msgb.at[0], acc.at[dstblk.at[jt]], sems0,
                         add=True)
        return 0
    lax.fori_loop(0, NBLK, _block, 0)

    pltpu.make_async_copy(msgb.at[0], acc.at[dstblk.at[0]], sems0).wait()
    pltpu.make_async_copy(msgb.at[1], acc.at[dstblk.at[0]], sems1).wait()
    plsc.subcore_barrier()

    for k in range(CPT):
        ch = sid + NS * k
        @pl.when(ch < NCHUNK)
        def _():
            pltpu.async_copy(acc.at[pl.ds(ch * ZCH, ZCH)],
                             out_hbm.at[cid, pl.ds(ch * ZCH, ZCH)], semg0)
    for k in range(CPT):
        ch = sid + NS * k
        @pl.when(ch < NCHUNK)
        def _():
            pltpu.make_async_copy(acc.at[pl.ds(ch * ZCH, ZCH)],
                                  out_hbm.at[cid, pl.ds(ch * ZCH, ZCH)],
                                  semg0).wait()


def _edge_pass(src, dst, xl, fg):
    mesh = plsc.VectorSubcoreMesh(core_axis_name="c", subcore_axis_name="s",
                                  num_cores=NC, num_subcores=NS)
    f = functools.partial(
        pl.kernel,
        out_type=jax.ShapeDtypeStruct((NC, N_NODES, COUT), jnp.float32),
        mesh=mesh,
        scratch_types=[
            pltpu.VMEM((BPB, B), jnp.int32),
            pltpu.VMEM((BPB, B), jnp.int32),
            pltpu.VMEM((2, B, CIN), jnp.float32),
            pltpu.VMEM((2, B, 2 * COUT), jnp.float32),
            pltpu.VMEM((2, B, COUT), jnp.float32),
            pltpu.VMEM_SHARED((N_NODES, COUT), jnp.float32),
            pltpu.SemaphoreType.DMA,
            pltpu.SemaphoreType.DMA,
            pltpu.SemaphoreType.DMA,
            pltpu.SemaphoreType.DMA,
        ],
    )(_edge_body)
    return f(src.reshape(NW, NBLK, BPB, B), dst.reshape(NW, NBLK, BPB, B),
             xl, fg)


def _combine_body(acc_ref, cnt_ref, outk_ref, o_ref):
    agg = acc_ref[0] + acc_ref[1]
    o_ref[...] = outk_ref[...] + agg / jnp.maximum(cnt_ref[...], 1.0)


def _combine(acc, cnt, outk):
    return pl.pallas_call(
        _combine_body,
        out_shape=jax.ShapeDtypeStruct((N_NODES, COUT), jnp.float32),
    )(acc, cnt, outk)


def kernel(x, edge_index, lin_w, film_w, film_b, lin_skip_w, film_skip_w):
    src = edge_index[0].astype(jnp.int32)
    dst = edge_index[1].astype(jnp.int32)
    xl, fg, outk = _dense(x, lin_w, film_w, film_b, lin_skip_w, film_skip_w)
    acc = _edge_pass(src, dst, xl, fg)
    hist = _degree(dst.reshape(N_EDGES, 1))
    cnt = hist.reshape(-1)[:N_NODES].reshape(N_NODES, 1)
    return _combine(acc, cnt, outk)
